# baseline (speedup 1.0000x reference)
import functools

import jax
import jax.numpy as jnp
from jax import lax
from jax.experimental import pallas as pl
from jax.experimental.pallas import tpu as pltpu
from jax.experimental.pallas import tpu_sc as plsc

PAD = 1
BSZ = 4
SEQ = 4096
D = 1024
G = BSZ * SEQ
NW = 32
JW = SEQ // NW
R = 32
NCH = JW // R
L = 16


def _make_kernel():
    mesh = plsc.VectorSubcoreMesh(core_axis_name="c", subcore_axis_name="s")

    @functools.partial(
        pl.kernel,
        mesh=mesh,
        out_type=jax.ShapeDtypeStruct((BSZ, SEQ, D), jnp.float32),
        scratch_types=[
            pltpu.VMEM((BSZ, JW), jnp.int32),
            pltpu.VMEM((R,), jnp.int32),
            pltpu.VMEM((R,), jnp.int32),
            pltpu.VMEM((R, D), jnp.float32),
            pltpu.VMEM((R, D), jnp.float32),
            pltpu.SemaphoreType.DMA,
            pltpu.SemaphoreType.DMA,
            pltpu.SemaphoreType.DMA,
            pltpu.SemaphoreType.DMA,
            pltpu.SemaphoreType.DMA,
        ],
    )
    def k(inp_hbm, w_hbm, out_hbm, tok_v, idx0, idx1, buf0, buf1,
          sem_r0, sem_r1, sem_w0, sem_w1, sem_t):
        wid = lax.axis_index("s") * 2 + lax.axis_index("c")
        jlo = wid * JW

        idxs = (idx0, idx1)
        bufs = (buf0, buf1)
        rsems = (sem_r0, sem_r1)
        wsems = (sem_w0, sem_w1)
        iota = lax.iota(jnp.int32, L)

        CH = ((0, 16), (16, 16), (32, 32), (64, 32), (96, 32))

        def gather_affine(i):
            start, size = CH[i]
            p = i % 2
            for v in range(size // L):
                idxs[p][pl.ds(v * L, L)] = (
                    jlo + start + v * L + (PAD + 1) + iota)
            src = w_hbm.at[idxs[p].at[pl.ds(0, size)] if size < R
                           else idxs[p]]
            dst = bufs[p].at[pl.ds(0, size)] if size < R else bufs[p]
            return pltpu.async_copy(src, dst, rsems[p])

        rcps = {0: gather_affine(0), 1: gather_affine(1)}
        tcp = pltpu.async_copy(inp_hbm.at[:, pl.ds(jlo, JW)], tok_v, sem_t)

        tcp.wait()
        GPB = JW // L

        def scan_body(g, acc):
            b = g // GPB
            tok = tok_v[b, pl.ds((g - b * GPB) * L, L)]
            return acc | jnp.where(tok == PAD, 1, 0)

        padv = lax.fori_loop(0, BSZ * GPB, scan_body,
                             jnp.zeros((L,), jnp.int32))
        anypad = padv[0]
        for l in range(1, L):
            anypad = anypad | padv[l]

        @pl.when(anypad == 0)
        def _fast():
            wcps = {}
            for c in range(len(CH)):
                start, size = CH[c]
                rcps[c].wait()
                src = bufs[c % 2].at[pl.ds(0, size)] if size < R \
                    else bufs[c % 2]
                wcps[c] = [
                    pltpu.async_copy(
                        src, out_hbm.at[b, pl.ds(jlo + start, size)],
                        wsems[c % 2])
                    for b in range(BSZ)
                ]
                if c + 2 < len(CH):
                    for cp in wcps[c]:
                        cp.wait()
                    rcps[c + 2] = gather_affine(c + 2)
            for c in range(len(CH) - 2, len(CH)):
                for cp in wcps[c]:
                    cp.wait()

        @pl.when(anypad != 0)
        def _slow():
            rcps[0].wait()
            rcps[1].wait()

            def step(s, carry):
                b = s // NCH
                c = s - b * NCH

                def fill(v, carry2):
                    tok = tok_v[b, pl.ds(c * R + v * L, L)]
                    pos = jlo + c * R + v * L + (PAD + 1) + iota
                    idx0[pl.ds(v * L, L)] = jnp.where(tok != PAD, pos, PAD)
                    return carry2

                lax.fori_loop(0, R // L, fill, jnp.int32(0))
                pltpu.async_copy(w_hbm.at[idx0], buf0, sem_r0).wait()
                dst = pl.multiple_of(jlo + c * R, 8)
                pltpu.async_copy(buf0, out_hbm.at[b, pl.ds(dst, R)],
                                 sem_w0).wait()
                return carry

            lax.fori_loop(0, BSZ * NCH, step, jnp.int32(0))

    return k


_embed = _make_kernel()


@jax.jit
def kernel(input, weights):
    return _embed(input, weights)


# speedup vs baseline: 1.0056x; 1.0056x over previous
import functools

import jax
import jax.numpy as jnp
from jax import lax
from jax.experimental import pallas as pl
from jax.experimental.pallas import tpu as pltpu
from jax.experimental.pallas import tpu_sc as plsc

PAD = 1
BSZ = 4
SEQ = 4096
D = 1024
G = BSZ * SEQ
NW = 32
JW = SEQ // NW
R = 32
NCH = JW // R
L = 16


def _make_kernel():
    mesh = plsc.VectorSubcoreMesh(core_axis_name="c", subcore_axis_name="s")

    @functools.partial(
        pl.kernel,
        mesh=mesh,
        out_type=jax.ShapeDtypeStruct((BSZ, SEQ, D), jnp.float32),
        scratch_types=[
            pltpu.VMEM((BSZ, JW), jnp.int32),
            pltpu.VMEM((R,), jnp.int32),
            pltpu.VMEM((R,), jnp.int32),
            pltpu.VMEM((R, D), jnp.float32),
            pltpu.VMEM((R, D), jnp.float32),
            pltpu.SemaphoreType.DMA,
            pltpu.SemaphoreType.DMA,
            pltpu.SemaphoreType.DMA,
            pltpu.SemaphoreType.DMA,
            pltpu.SemaphoreType.DMA,
        ],
    )
    def k(inp_hbm, w_hbm, out_hbm, tok_v, idx0, idx1, buf0, buf1,
          sem_r0, sem_r1, sem_w0, sem_w1, sem_t):
        wid = lax.axis_index("s") * 2 + lax.axis_index("c")
        jlo = wid * JW

        idxs = (idx0, idx1)
        bufs = (buf0, buf1)
        rsems = (sem_r0, sem_r1)
        wsems = (sem_w0, sem_w1)
        iota = lax.iota(jnp.int32, L)

        def gather_affine(c):
            p = c % 2
            for v in range(R // L):
                idxs[p][pl.ds(v * L, L)] = (
                    jlo + c * R + v * L + (PAD + 1) + iota)
            return pltpu.async_copy(w_hbm.at[idxs[p]], bufs[p], rsems[p])

        def bcast_write(c):
            return [
                pltpu.async_copy(bufs[c % 2],
                                 out_hbm.at[b, pl.ds(jlo + c * R, R)],
                                 wsems[c % 2])
                for b in range(BSZ)
            ]

        rcps = {0: gather_affine(0), 1: gather_affine(1)}
        tcp = pltpu.async_copy(inp_hbm.at[:, pl.ds(jlo, JW)], tok_v, sem_t)
        wcps = {}
        for c in (0, 1):
            rcps[c].wait()
            wcps[c] = bcast_write(c)

        tcp.wait()
        GPB = JW // L

        def scan_body(g, acc):
            b = g // GPB
            tok = tok_v[b, pl.ds((g - b * GPB) * L, L)]
            return acc | jnp.where(tok == PAD, 1, 0)

        padv = lax.fori_loop(0, BSZ * GPB, scan_body,
                             jnp.zeros((L,), jnp.int32))
        anypad = padv[0]
        for l in range(1, L):
            anypad = anypad | padv[l]

        @pl.when(anypad == 0)
        def _fast():
            for c in range(2, NCH):
                for cp in wcps[c - 2]:
                    cp.wait()
                rcps[c] = gather_affine(c)
                rcps[c].wait()
                wcps[c] = bcast_write(c)
            for c in range(NCH - 2, NCH):
                for cp in wcps[c]:
                    cp.wait()

        @pl.when(anypad != 0)
        def _slow():
            for c in (0, 1):
                for cp in wcps[c]:
                    cp.wait()

            def step(s, carry):
                b = s // NCH
                c = s - b * NCH

                def fill(v, carry2):
                    tok = tok_v[b, pl.ds(c * R + v * L, L)]
                    pos = jlo + c * R + v * L + (PAD + 1) + iota
                    idx0[pl.ds(v * L, L)] = jnp.where(tok != PAD, pos, PAD)
                    return carry2

                lax.fori_loop(0, R // L, fill, jnp.int32(0))
                pltpu.async_copy(w_hbm.at[idx0], buf0, sem_r0).wait()
                dst = pl.multiple_of(jlo + c * R, 8)
                pltpu.async_copy(buf0, out_hbm.at[b, pl.ds(dst, R)],
                                 sem_w0).wait()
                return carry

            lax.fori_loop(0, BSZ * NCH, step, jnp.int32(0))

    return k


_embed = _make_kernel()


@jax.jit
def kernel(input, weights):
    return _embed(input, weights)
